# Initial kernel scaffold; baseline (speedup 1.0000x reference)
import jax
import jax.numpy as jnp
from jax.experimental import pallas as pl


def kernel(x, adj, edge_index, W_lin, b_lin, W_self1, W_neigh1, b1, W_self2, W_neigh2, b2):
    raise NotImplementedError("write your pallas kernel here")


# speedup vs baseline: 5.2440x; 5.2440x over previous
import functools

import jax
import jax.numpy as jnp
from jax import lax
from jax.experimental import pallas as pl
from jax.experimental.pallas import tpu as pltpu
from jax.experimental.pallas import tpu_sc as plsc

N = 10000
E = 320000
ALPHA = 0.2

NC = 2
NS = 16
NW = NC * NS
EDGES_PER_TILE = E // NW
K = 80
CHUNKS = EDGES_PER_TILE // K
SLICE = 624
REM = N - NS * SLICE
DEGW = 16


def _leaky(v):
    return jnp.where(v >= 0, v, ALPHA * v)


def _matT(a, w):
    return lax.dot_general(a, w, (((1,), (1,)), ((), ())),
                           preferred_element_type=jnp.float32)


def _make_seg_sum(d, with_deg):
    mesh = plsc.VectorSubcoreMesh(core_axis_name="c", subcore_axis_name="s")
    out_type = [jax.ShapeDtypeStruct((NC * N, d), jnp.float32)]
    scratch = [
        pltpu.VMEM((K,), jnp.int32),
        pltpu.VMEM((K,), jnp.int32),
        pltpu.VMEM((K, d), jnp.float32),
        pltpu.VMEM_SHARED((N, d), jnp.float32),
        pltpu.SemaphoreType.DMA,
    ]
    if with_deg:
        out_type.append(jax.ShapeDtypeStruct((NC * N, DEGW), jnp.float32))
        scratch += [
            pltpu.VMEM((K, DEGW), jnp.float32),
            pltpu.VMEM_SHARED((N, DEGW), jnp.float32),
        ]

    def body(h_hbm, src_hbm, dst_hbm, z_hbm, zd_hbm, p_hbm, dp_hbm,
             src_idx, dst_idx, rows, acc, sem, ones, dacc):
        c = lax.axis_index("c")
        s = lax.axis_index("s")
        wid = c * NS + s
        r0 = s * SLICE

        pltpu.sync_copy(z_hbm.at[pl.ds(r0, SLICE)], acc.at[pl.ds(r0, SLICE)])

        @pl.when(s == 0)
        def _():
            pltpu.sync_copy(z_hbm.at[pl.ds(NS * SLICE, REM)],
                            acc.at[pl.ds(NS * SLICE, REM)])

        if with_deg:
            pltpu.sync_copy(zd_hbm.at[pl.ds(r0, SLICE)],
                            dacc.at[pl.ds(r0, SLICE)])

            @pl.when(s == 0)
            def _():
                pltpu.sync_copy(zd_hbm.at[pl.ds(NS * SLICE, REM)],
                                dacc.at[pl.ds(NS * SLICE, REM)])

            for i in range(K):
                ones[i, pl.ds(0, 16)] = jnp.ones((16,), jnp.float32)
        plsc.subcore_barrier()

        ebase = wid * EDGES_PER_TILE

        def chunk(i, _):
            b = ebase + i * K
            pltpu.sync_copy(src_hbm.at[pl.ds(b, K)], src_idx)
            pltpu.sync_copy(dst_hbm.at[pl.ds(b, K)], dst_idx)
            pltpu.async_copy(h_hbm.at[src_idx], rows, sem).wait()
            pltpu.sync_copy(rows, acc.at[dst_idx], add=True)
            if with_deg:
                pltpu.sync_copy(ones, dacc.at[dst_idx], add=True)
            return ()

        lax.fori_loop(0, CHUNKS, chunk, (), unroll=False)
        plsc.subcore_barrier()

        o0 = c * N + r0
        pltpu.sync_copy(acc.at[pl.ds(r0, SLICE)], p_hbm.at[pl.ds(o0, SLICE)])

        @pl.when(s == 0)
        def _():
            pltpu.sync_copy(acc.at[pl.ds(NS * SLICE, REM)],
                            p_hbm.at[pl.ds(c * N + NS * SLICE, REM)])

        if with_deg:
            pltpu.sync_copy(dacc.at[pl.ds(r0, SLICE)],
                            dp_hbm.at[pl.ds(o0, SLICE)])

            @pl.when(s == 0)
            def _():
                pltpu.sync_copy(dacc.at[pl.ds(NS * SLICE, REM)],
                                dp_hbm.at[pl.ds(c * N + NS * SLICE, REM)])

    if with_deg:
        def body_wrap(h, src, dst, z, zd, p, dp, src_idx, dst_idx, rows, acc,
                      sem, ones, dacc):
            body(h, src, dst, z, zd, p, dp, src_idx, dst_idx, rows, acc, sem,
                 ones, dacc)
    else:
        def body_wrap(h, src, dst, z, p, src_idx, dst_idx, rows, acc, sem):
            body(h, src, dst, z, None, p, None, src_idx, dst_idx, rows, acc,
                 sem, None, None)

    return pl.kernel(body_wrap, out_type=tuple(out_type), mesh=mesh,
                     scratch_types=scratch,
                     compiler_params=pltpu.CompilerParams(
                         use_tc_tiling_on_sc=False))


_seg_sum_deg_128 = _make_seg_sum(128, True)
_seg_sum_64 = _make_seg_sum(64, False)


def _stem_body(x_ref, w_ref, b_ref, o_ref):
    o_ref[...] = _leaky(_matT(x_ref[...], w_ref[...]) + b_ref[...])


def _layer1_body(h_ref, p_ref, dp_ref, ws_ref, wn_ref, b_ref, wn2_ref,
                 o1_ref, hw2_ref):
    ssum = p_ref[0:N, :] + p_ref[N:2 * N, :]
    deg = dp_ref[0:N, 0:1] + dp_ref[N:2 * N, 0:1]
    hn = ssum / jnp.maximum(deg, 1.0)
    o1 = _leaky(_matT(h_ref[...], ws_ref[...]) + _matT(hn, wn_ref[...])
                + b_ref[...])
    o1_ref[...] = o1
    hw2_ref[...] = _matT(o1, wn2_ref[...])


def _layer2_body(o1_ref, q_ref, dp_ref, ws_ref, b_ref, o_ref):
    ssum = q_ref[0:N, :] + q_ref[N:2 * N, :]
    deg = dp_ref[0:N, 0:1] + dp_ref[N:2 * N, 0:1]
    t = _matT(o1_ref[...], ws_ref[...]) + ssum / jnp.maximum(deg, 1.0) \
        + b_ref[...]
    z = t - jnp.max(t, axis=1, keepdims=True)
    o_ref[...] = z - jnp.log(jnp.sum(jnp.exp(z), axis=1, keepdims=True))


def kernel(x, adj, edge_index, W_lin, b_lin, W_self1, W_neigh1, b1,
           W_self2, W_neigh2, b2):
    del adj
    src = edge_index[0]
    dst = edge_index[1]
    z128 = jnp.zeros((N, 128), jnp.float32)
    z64 = jnp.zeros((N, 64), jnp.float32)
    zd = jnp.zeros((N, DEGW), jnp.float32)

    h = pl.pallas_call(
        _stem_body,
        out_shape=jax.ShapeDtypeStruct((N, 128), jnp.float32),
    )(x, W_lin, b_lin.reshape(1, -1))

    p, dp = _seg_sum_deg_128(h, src, dst, z128, zd)

    o1, hw2 = pl.pallas_call(
        _layer1_body,
        out_shape=(jax.ShapeDtypeStruct((N, 128), jnp.float32),
                   jax.ShapeDtypeStruct((N, 64), jnp.float32)),
    )(h, p, dp, W_self1, W_neigh1, b1.reshape(1, -1), W_neigh2)

    (q,) = _seg_sum_64(hw2, src, dst, z64)

    out = pl.pallas_call(
        _layer2_body,
        out_shape=jax.ShapeDtypeStruct((N, 64), jnp.float32),
    )(o1, q, dp, W_self2, b2.reshape(1, -1))
    return out


# speedup vs baseline: 11.4880x; 2.1907x over previous
import functools

import jax
import jax.numpy as jnp
from jax import lax
from jax.experimental import pallas as pl
from jax.experimental.pallas import tpu as pltpu
from jax.experimental.pallas import tpu_sc as plsc

N = 10000
E = 320000
ALPHA = 0.2

NC = 2
NS = 16
NW = NC * NS
EDGES_PER_TILE = E // NW
K = 80
CHUNKS = EDGES_PER_TILE // K
SLICE = 624
REM = N - NS * SLICE
DEGW = 8


def _leaky(v):
    return jnp.where(v >= 0, v, ALPHA * v)


def _matT(a, w):
    return lax.dot_general(a, w, (((1,), (1,)), ((), ())),
                           preferred_element_type=jnp.float32)


def _make_seg_sum(d, with_deg):
    mesh = plsc.VectorSubcoreMesh(core_axis_name="c", subcore_axis_name="s")
    out_type = [jax.ShapeDtypeStruct((NC * N, d), jnp.float32)]
    scratch = [
        pltpu.VMEM((CHUNKS, K), jnp.int32),
        pltpu.VMEM((CHUNKS, K), jnp.int32),
        pltpu.VMEM((K, d), jnp.float32),
        pltpu.VMEM((K, d), jnp.float32),
        pltpu.VMEM_SHARED((N, d), jnp.float32),
        pltpu.SemaphoreType.DMA,
        pltpu.SemaphoreType.DMA,
    ]
    if with_deg:
        out_type.append(jax.ShapeDtypeStruct((NC * N, DEGW), jnp.float32))
        scratch += [
            pltpu.VMEM((K, DEGW), jnp.float32),
            pltpu.VMEM_SHARED((N, DEGW), jnp.float32),
        ]

    def body(h_hbm, src_hbm, dst_hbm, z_hbm, zd_hbm, ones_hbm, p_hbm,
             dp_hbm, src_all, dst_all, buf0, buf1, acc, sem0, sem1, ones,
             dacc):
        c = lax.axis_index("c")
        s = lax.axis_index("s")
        wid = c * NS + s
        r0 = s * SLICE

        pltpu.sync_copy(z_hbm.at[pl.ds(r0, SLICE)], acc.at[pl.ds(r0, SLICE)])

        @pl.when(s == 0)
        def _():
            pltpu.sync_copy(z_hbm.at[pl.ds(NS * SLICE, REM)],
                            acc.at[pl.ds(NS * SLICE, REM)])

        if with_deg:
            pltpu.sync_copy(zd_hbm.at[pl.ds(r0, SLICE)],
                            dacc.at[pl.ds(r0, SLICE)])

            @pl.when(s == 0)
            def _():
                pltpu.sync_copy(zd_hbm.at[pl.ds(NS * SLICE, REM)],
                                dacc.at[pl.ds(NS * SLICE, REM)])

            pltpu.sync_copy(ones_hbm, ones)
        plsc.subcore_barrier()

        pltpu.sync_copy(src_hbm.at[wid], src_all)
        pltpu.sync_copy(dst_hbm.at[wid], dst_all)

        def launch(i, buf, sem):
            pltpu.async_copy(h_hbm.at[src_all.at[i]], buf, sem)

        def consume(i, buf, sem):
            pltpu.make_async_copy(h_hbm.at[src_all.at[i]], buf, sem).wait()
            pltpu.sync_copy(buf, acc.at[dst_all.at[i]], add=True)
            if with_deg:
                pltpu.sync_copy(ones, dacc.at[dst_all.at[i]], add=True)

        launch(0, buf0, sem0)

        def pair(j, _):
            launch(2 * j + 1, buf1, sem1)
            consume(2 * j, buf0, sem0)
            launch(2 * j + 2, buf0, sem0)
            consume(2 * j + 1, buf1, sem1)
            return ()

        lax.fori_loop(0, CHUNKS // 2, pair, (), unroll=False)
        consume(CHUNKS - 1, buf0, sem0)
        plsc.subcore_barrier()

        o0 = c * N + r0
        pltpu.sync_copy(acc.at[pl.ds(r0, SLICE)], p_hbm.at[pl.ds(o0, SLICE)])

        @pl.when(s == 0)
        def _():
            pltpu.sync_copy(acc.at[pl.ds(NS * SLICE, REM)],
                            p_hbm.at[pl.ds(c * N + NS * SLICE, REM)])

        if with_deg:
            pltpu.sync_copy(dacc.at[pl.ds(r0, SLICE)],
                            dp_hbm.at[pl.ds(o0, SLICE)])

            @pl.when(s == 0)
            def _():
                pltpu.sync_copy(dacc.at[pl.ds(NS * SLICE, REM)],
                                dp_hbm.at[pl.ds(c * N + NS * SLICE, REM)])

    if with_deg:
        def body_wrap(h, src, dst, z, zd, ones_in, p, dp, src_all, dst_all,
                      buf0, buf1, acc, sem0, sem1, ones, dacc):
            body(h, src, dst, z, zd, ones_in, p, dp, src_all, dst_all, buf0,
                 buf1, acc, sem0, sem1, ones, dacc)
    else:
        def body_wrap(h, src, dst, z, p, src_all, dst_all, buf0, buf1, acc,
                      sem0, sem1):
            body(h, src, dst, z, None, None, p, None, src_all, dst_all,
                 buf0, buf1, acc, sem0, sem1, None, None)

    return pl.kernel(body_wrap, out_type=tuple(out_type), mesh=mesh,
                     scratch_types=scratch,
                     compiler_params=pltpu.CompilerParams(
                         use_tc_tiling_on_sc=False))


_seg_sum_deg_128 = _make_seg_sum(128, True)
_seg_sum_64 = _make_seg_sum(64, False)


def _stem_body(x_ref, w_ref, b_ref, o_ref):
    o_ref[...] = _leaky(_matT(x_ref[...], w_ref[...]) + b_ref[...])


def _layer1_body(h_ref, p_ref, dp_ref, ws_ref, wn_ref, b_ref, wn2_ref,
                 o1_ref, hw2_ref):
    ssum = p_ref[0:N, :] + p_ref[N:2 * N, :]
    deg = dp_ref[0:N, 0:1] + dp_ref[N:2 * N, 0:1]
    hn = ssum / jnp.maximum(deg, 1.0)
    o1 = _leaky(_matT(h_ref[...], ws_ref[...]) + _matT(hn, wn_ref[...])
                + b_ref[...])
    o1_ref[...] = o1
    hw2_ref[...] = _matT(o1, wn2_ref[...])


def _layer2_body(o1_ref, q_ref, dp_ref, ws_ref, b_ref, o_ref):
    ssum = q_ref[0:N, :] + q_ref[N:2 * N, :]
    deg = dp_ref[0:N, 0:1] + dp_ref[N:2 * N, 0:1]
    t = _matT(o1_ref[...], ws_ref[...]) + ssum / jnp.maximum(deg, 1.0) \
        + b_ref[...]
    z = t - jnp.max(t, axis=1, keepdims=True)
    o_ref[...] = z - jnp.log(jnp.sum(jnp.exp(z), axis=1, keepdims=True))


def kernel(x, adj, edge_index, W_lin, b_lin, W_self1, W_neigh1, b1,
           W_self2, W_neigh2, b2):
    del adj
    src = edge_index[0].reshape(NW, CHUNKS, K)
    dst = edge_index[1].reshape(NW, CHUNKS, K)
    z128 = jnp.zeros((N, 128), jnp.float32)
    z64 = jnp.zeros((N, 64), jnp.float32)
    zd = jnp.zeros((N, DEGW), jnp.float32)

    h = pl.pallas_call(
        _stem_body,
        out_shape=jax.ShapeDtypeStruct((N, 128), jnp.float32),
    )(x, W_lin, b_lin.reshape(1, -1))

    ones_in = jnp.ones((K, DEGW), jnp.float32)
    p, dp = _seg_sum_deg_128(h, src, dst, z128, zd, ones_in)

    o1, hw2 = pl.pallas_call(
        _layer1_body,
        out_shape=(jax.ShapeDtypeStruct((N, 128), jnp.float32),
                   jax.ShapeDtypeStruct((N, 64), jnp.float32)),
    )(h, p, dp, W_self1, W_neigh1, b1.reshape(1, -1), W_neigh2)

    (q,) = _seg_sum_64(hw2, src, dst, z64)

    out = pl.pallas_call(
        _layer2_body,
        out_shape=jax.ShapeDtypeStruct((N, 64), jnp.float32),
    )(o1, q, dp, W_self2, b2.reshape(1, -1))
    return out
